# Initial kernel scaffold; baseline (speedup 1.0000x reference)
import jax
import jax.numpy as jnp
from jax.experimental import pallas as pl


def kernel(user_embeds_list, userIdx, W1, b1, W2, b2):
    raise NotImplementedError("write your pallas kernel here")


# speedup vs baseline: 1.0731x; 1.0731x over previous
import functools

import jax
import jax.numpy as jnp
from jax.experimental import pallas as pl


def _agg_kernel(e_ref, w1_ref, b1_ref, w2_ref, b2_ref, o_ref):
    S = e_ref.shape[0]
    w1 = w1_ref[...]
    b1 = b1_ref[...]
    w2 = w2_ref[...]
    b2 = b2_ref[0, 0]

    slices = []
    logits = []
    for s in range(S):
        e = e_ref[s]
        h = jnp.maximum(
            jnp.dot(e, w1, preferred_element_type=jnp.float32) + b1, 0.0)
        logit = jnp.sum(h * w2, axis=1, keepdims=True) + b2
        slices.append(e)
        logits.append(logit)

    m = logits[0]
    for s in range(1, S):
        m = jnp.maximum(m, logits[s])
    exps = [jnp.exp(l - m) for l in logits]
    denom = exps[0]
    for s in range(1, S):
        denom = denom + exps[s]
    acc = exps[0] * slices[0]
    for s in range(1, S):
        acc = acc + exps[s] * slices[s]
    o_ref[...] = acc / denom


@functools.partial(jax.jit, static_argnames=("interpret",))
def kernel(user_embeds_list, userIdx, W1, b1, W2, b2, interpret=False):
    del userIdx
    S, B, D = user_embeds_list.shape
    H = W1.shape[1]
    bblk = min(B, 512)

    return pl.pallas_call(
        _agg_kernel,
        grid=(B // bblk,),
        in_specs=[
            pl.BlockSpec((S, bblk, D), lambda i: (0, i, 0)),
            pl.BlockSpec((D, H), lambda i: (0, 0)),
            pl.BlockSpec((1, H), lambda i: (0, 0)),
            pl.BlockSpec((1, H), lambda i: (0, 0)),
            pl.BlockSpec((1, 1), lambda i: (0, 0)),
        ],
        out_specs=pl.BlockSpec((bblk, D), lambda i: (i, 0)),
        out_shape=jax.ShapeDtypeStruct((B, D), jnp.float32),
        interpret=interpret,
    )(
        user_embeds_list.astype(jnp.float32),
        W1.astype(jnp.float32),
        b1.reshape(1, H).astype(jnp.float32),
        W2.reshape(1, H).astype(jnp.float32),
        b2.reshape(1, 1).astype(jnp.float32),
    )


# speedup vs baseline: 1.2633x; 1.1773x over previous
import functools

import jax
import jax.numpy as jnp
from jax.experimental import pallas as pl


def _agg_kernel(e_ref, w1_ref, b1_ref, w2_ref, b2_ref, o_ref):
    S = e_ref.shape[0]
    w1 = w1_ref[...]
    b1 = b1_ref[...]
    w2 = w2_ref[...]
    b2 = b2_ref[0, 0]

    slices = []
    logits = []
    for s in range(S):
        e = e_ref[s]
        h = jnp.maximum(
            jnp.dot(e, w1, preferred_element_type=jnp.float32) + b1, 0.0)
        logit = jnp.sum(h * w2, axis=1, keepdims=True) + b2
        slices.append(e)
        logits.append(logit)

    m = logits[0]
    for s in range(1, S):
        m = jnp.maximum(m, logits[s])
    exps = [jnp.exp(l - m) for l in logits]
    denom = exps[0]
    for s in range(1, S):
        denom = denom + exps[s]
    acc = exps[0] * slices[0]
    for s in range(1, S):
        acc = acc + exps[s] * slices[s]
    o_ref[...] = acc / denom


@functools.partial(jax.jit, static_argnames=("interpret",))
def kernel(user_embeds_list, userIdx, W1, b1, W2, b2, interpret=False):
    del userIdx
    S, B, D = user_embeds_list.shape
    H = W1.shape[1]
    bblk = min(B, 1024)

    return pl.pallas_call(
        _agg_kernel,
        grid=(B // bblk,),
        in_specs=[
            pl.BlockSpec((S, bblk, D), lambda i: (0, i, 0)),
            pl.BlockSpec((D, H), lambda i: (0, 0)),
            pl.BlockSpec((1, H), lambda i: (0, 0)),
            pl.BlockSpec((1, H), lambda i: (0, 0)),
            pl.BlockSpec((1, 1), lambda i: (0, 0)),
        ],
        out_specs=pl.BlockSpec((bblk, D), lambda i: (i, 0)),
        out_shape=jax.ShapeDtypeStruct((B, D), jnp.float32),
        interpret=interpret,
    )(
        user_embeds_list.astype(jnp.float32),
        W1.astype(jnp.float32),
        b1.reshape(1, H).astype(jnp.float32),
        W2.reshape(1, H).astype(jnp.float32),
        b2.reshape(1, 1).astype(jnp.float32),
    )


# speedup vs baseline: 1.3005x; 1.0294x over previous
import functools

import jax
import jax.numpy as jnp
from jax.experimental import pallas as pl


def _agg_kernel(e_ref, w1_ref, b1_ref, w2_ref, b2_ref, o_ref):
    S = e_ref.shape[0]
    w1 = w1_ref[...]
    b1 = b1_ref[...]
    w2 = w2_ref[...]
    b2 = b2_ref[0, 0]

    slices = []
    logits = []
    for s in range(S):
        e = e_ref[s]
        h = jnp.maximum(
            jnp.dot(e, w1, preferred_element_type=jnp.float32) + b1, 0.0)
        logit = jnp.sum(h * w2, axis=1, keepdims=True) + b2
        slices.append(e)
        logits.append(logit)

    m = logits[0]
    for s in range(1, S):
        m = jnp.maximum(m, logits[s])
    exps = [jnp.exp(l - m) for l in logits]
    denom = exps[0]
    for s in range(1, S):
        denom = denom + exps[s]
    acc = exps[0] * slices[0]
    for s in range(1, S):
        acc = acc + exps[s] * slices[s]
    o_ref[...] = acc / denom


@functools.partial(jax.jit, static_argnames=("interpret",))
def kernel(user_embeds_list, userIdx, W1, b1, W2, b2, interpret=False):
    del userIdx
    S, B, D = user_embeds_list.shape
    H = W1.shape[1]
    bblk = min(B, 2048)

    return pl.pallas_call(
        _agg_kernel,
        grid=(B // bblk,),
        in_specs=[
            pl.BlockSpec((S, bblk, D), lambda i: (0, i, 0)),
            pl.BlockSpec((D, H), lambda i: (0, 0)),
            pl.BlockSpec((1, H), lambda i: (0, 0)),
            pl.BlockSpec((1, H), lambda i: (0, 0)),
            pl.BlockSpec((1, 1), lambda i: (0, 0)),
        ],
        out_specs=pl.BlockSpec((bblk, D), lambda i: (i, 0)),
        out_shape=jax.ShapeDtypeStruct((B, D), jnp.float32),
        interpret=interpret,
    )(
        user_embeds_list.astype(jnp.float32),
        W1.astype(jnp.float32),
        b1.reshape(1, H).astype(jnp.float32),
        W2.reshape(1, H).astype(jnp.float32),
        b2.reshape(1, 1).astype(jnp.float32),
    )
